# initial kernel scaffold (unmeasured)
import jax
import jax.numpy as jnp
from jax import lax
from jax.experimental import pallas as pl
from jax.experimental.pallas import tpu as pltpu


def kernel(
    x,
):
    def body(*refs):
        pass

    out_shape = jax.ShapeDtypeStruct(..., jnp.float32)
    return pl.pallas_call(body, out_shape=out_shape)(...)


# baseline (device time: 155011 ns/iter reference)
import jax
import jax.numpy as jnp
from jax import lax
from jax.experimental import pallas as pl
from jax.experimental.pallas import tpu as pltpu

N_DEV = 4


def kernel(x):
    m, n = x.shape
    chunk = m // N_DEV

    def body(x_ref, out_ref, rs_buf, send_sems, recv_sems):
        my = lax.axis_index("i")
        left = (my + N_DEV - 1) % N_DEV
        right = (my + 1) % N_DEV

        barrier_sem = pltpu.get_barrier_semaphore()
        for nbr in [left, right]:
            pl.semaphore_signal(
                barrier_sem, inc=1,
                device_id=(nbr,), device_id_type=pl.DeviceIdType.MESH,
            )
        pl.semaphore_wait(barrier_sem, 2)


        rdma = pltpu.make_async_remote_copy(
            src_ref=x_ref.at[pl.ds(my * chunk, chunk), :],
            dst_ref=rs_buf.at[0],
            send_sem=send_sems.at[0],
            recv_sem=recv_sems.at[0],
            device_id=(right,),
            device_id_type=pl.DeviceIdType.MESH,
        )
        rdma.start()
        rdma.wait()
        c = (my + N_DEV - 1) % N_DEV
        rs_buf[0, :, :] = rs_buf[0, :, :] + x_ref[pl.ds(c * chunk, chunk), :]

        for s in range(1, N_DEV - 1):
            rdma = pltpu.make_async_remote_copy(
                src_ref=rs_buf.at[s - 1],
                dst_ref=rs_buf.at[s],
                send_sem=send_sems.at[s],
                recv_sem=recv_sems.at[s],
                device_id=(right,),
                device_id_type=pl.DeviceIdType.MESH,
            )
            rdma.start()
            rdma.wait()
            c = (my + N_DEV - s - 1) % N_DEV
            if s < N_DEV - 2:
                rs_buf[s, :, :] = (
                    rs_buf[s, :, :] + x_ref[pl.ds(c * chunk, chunk), :]
                )

        q = (my + 1) % N_DEV
        out_ref[pl.ds(q * chunk, chunk), :] = (
            rs_buf[N_DEV - 2, :, :] + x_ref[pl.ds(q * chunk, chunk), :]
        )

        for t in range(N_DEV - 1):
            c_t = (my + 1 + N_DEV - t) % N_DEV
            rdma = pltpu.make_async_remote_copy(
                src_ref=out_ref.at[pl.ds(c_t * chunk, chunk), :],
                dst_ref=out_ref.at[pl.ds(c_t * chunk, chunk), :],
                send_sem=send_sems.at[N_DEV - 1 + t],
                recv_sem=recv_sems.at[N_DEV - 1 + t],
                device_id=(right,),
                device_id_type=pl.DeviceIdType.MESH,
            )
            rdma.start()
            rdma.wait()

    return pl.pallas_call(
        body,
        out_shape=jax.ShapeDtypeStruct((m, n), x.dtype),
        in_specs=[pl.BlockSpec(memory_space=pltpu.VMEM)],
        out_specs=pl.BlockSpec(memory_space=pltpu.VMEM),
        scratch_shapes=[
            pltpu.VMEM((N_DEV - 1, chunk, n), x.dtype),
            pltpu.SemaphoreType.DMA((2 * (N_DEV - 1),)),
            pltpu.SemaphoreType.DMA((2 * (N_DEV - 1),)),
        ],
        compiler_params=pltpu.CompilerParams(collective_id=0),
    )(x)


# device time: 87810 ns/iter; 1.7653x vs baseline; 1.7653x over previous
import jax
import jax.numpy as jnp
from jax import lax
from jax.experimental import pallas as pl
from jax.experimental.pallas import tpu as pltpu

N_DEV = 4


def kernel(x):
    m, n = x.shape
    chunk = m // N_DEV
    n2 = n // 2

    def body(x_ref, out_ref, rs_r, rs_l, send_sems, recv_sems):
        my = lax.axis_index("i")
        left = (my + N_DEV - 1) % N_DEV
        right = (my + 1) % N_DEV

        barrier_sem = pltpu.get_barrier_semaphore()
        for nbr in [left, right]:
            pl.semaphore_signal(
                barrier_sem, inc=1,
                device_id=(nbr,), device_id_type=pl.DeviceIdType.MESH,
            )
        pl.semaphore_wait(barrier_sem, 2)

        def copy(src, dst, sem_idx, dev):
            return pltpu.make_async_remote_copy(
                src_ref=src, dst_ref=dst,
                send_sem=send_sems.at[sem_idx],
                recv_sem=recv_sems.at[sem_idx],
                device_id=(dev,), device_id_type=pl.DeviceIdType.MESH,
            )


        r = copy(x_ref.at[pl.ds(my * chunk, chunk), pl.ds(0, n2)],
                 rs_r.at[0], 0, right)
        l = copy(x_ref.at[pl.ds(my * chunk, chunk), pl.ds(n2, n2)],
                 rs_l.at[0], 1, left)
        r.start()
        l.start()
        r.wait()
        l.wait()
        cr = (my + N_DEV - 1) % N_DEV
        cl = (my + 1) % N_DEV
        rs_r[0, :, :] = rs_r[0, :, :] + x_ref[pl.ds(cr * chunk, chunk), pl.ds(0, n2)]
        rs_l[0, :, :] = rs_l[0, :, :] + x_ref[pl.ds(cl * chunk, chunk), pl.ds(n2, n2)]

        for s in range(1, N_DEV - 1):
            r = copy(rs_r.at[s - 1], rs_r.at[s], 2 * s, right)
            l = copy(rs_l.at[s - 1], rs_l.at[s], 2 * s + 1, left)
            r.start()
            l.start()
            r.wait()
            l.wait()
            cr = (my + N_DEV - s - 1) % N_DEV
            cl = (my + s + 1) % N_DEV
            if s < N_DEV - 2:
                rs_r[s, :, :] = (
                    rs_r[s, :, :] + x_ref[pl.ds(cr * chunk, chunk), pl.ds(0, n2)]
                )
                rs_l[s, :, :] = (
                    rs_l[s, :, :] + x_ref[pl.ds(cl * chunk, chunk), pl.ds(n2, n2)]
                )

        qr = (my + 1) % N_DEV
        ql = (my + N_DEV - 1) % N_DEV
        out_ref[pl.ds(qr * chunk, chunk), pl.ds(0, n2)] = (
            rs_r[N_DEV - 2, :, :]
            + x_ref[pl.ds(qr * chunk, chunk), pl.ds(0, n2)]
        )
        out_ref[pl.ds(ql * chunk, chunk), pl.ds(n2, n2)] = (
            rs_l[N_DEV - 2, :, :]
            + x_ref[pl.ds(ql * chunk, chunk), pl.ds(n2, n2)]
        )

        base = 2 * (N_DEV - 1)
        for t in range(N_DEV - 1):
            cr = (my + 1 + N_DEV - t) % N_DEV
            cl = (my + N_DEV - 1 + t) % N_DEV
            r = copy(out_ref.at[pl.ds(cr * chunk, chunk), pl.ds(0, n2)],
                     out_ref.at[pl.ds(cr * chunk, chunk), pl.ds(0, n2)],
                     base + 2 * t, right)
            l = copy(out_ref.at[pl.ds(cl * chunk, chunk), pl.ds(n2, n2)],
                     out_ref.at[pl.ds(cl * chunk, chunk), pl.ds(n2, n2)],
                     base + 2 * t + 1, left)
            r.start()
            l.start()
            r.wait()
            l.wait()

    n_sems = 4 * (N_DEV - 1)
    return pl.pallas_call(
        body,
        out_shape=jax.ShapeDtypeStruct((m, n), x.dtype),
        in_specs=[pl.BlockSpec(memory_space=pltpu.VMEM)],
        out_specs=pl.BlockSpec(memory_space=pltpu.VMEM),
        scratch_shapes=[
            pltpu.VMEM((N_DEV - 1, chunk, n2), x.dtype),
            pltpu.VMEM((N_DEV - 1, chunk, n2), x.dtype),
            pltpu.SemaphoreType.DMA((n_sems,)),
            pltpu.SemaphoreType.DMA((n_sems,)),
        ],
        compiler_params=pltpu.CompilerParams(collective_id=0),
    )(x)


# device time: 79390 ns/iter; 1.9525x vs baseline; 1.1061x over previous
import jax
import jax.numpy as jnp
from jax import lax
from jax.experimental import pallas as pl
from jax.experimental.pallas import tpu as pltpu

N_DEV = 4
K = 2


def kernel(x):
    m, n = x.shape
    chunk = m // N_DEV
    n2 = n // 2
    sub = n2 // K
    n_hops = 2 * (N_DEV - 1)
    n_sems = n_hops * 2 * K

    def body(x_ref, out_ref, rs_r, rs_l, send_sems, recv_sems):
        my = lax.axis_index("i")
        left = (my + N_DEV - 1) % N_DEV
        right = (my + 1) % N_DEV

        barrier_sem = pltpu.get_barrier_semaphore()
        for nbr in [left, right]:
            pl.semaphore_signal(
                barrier_sem, inc=1,
                device_id=(nbr,), device_id_type=pl.DeviceIdType.MESH,
            )
        pl.semaphore_wait(barrier_sem, 2)

        def dev(d):
            return right if d == 0 else left

        def gcol(d, c):
            return d * n2 + c * sub

        def rs_buf(d):
            return rs_r if d == 0 else rs_l

        def rs_recv_chunk(d, h):
            return ((my + N_DEV - h - 1) if d == 0 else (my + h + 1)) % N_DEV

        def owned_chunk(d):
            return ((my + 1) if d == 0 else (my + N_DEV - 1)) % N_DEV

        def ag_send_chunk(d, t):
            return ((my + 1 + N_DEV - t) if d == 0 else (my + N_DEV - 1 + t)) % N_DEV

        def sem(h, d, c):
            return (h * 2 + d) * K + c

        def copy(src, dst, h, d, c):
            return pltpu.make_async_remote_copy(
                src_ref=src, dst_ref=dst,
                send_sem=send_sems.at[sem(h, d, c)],
                recv_sem=recv_sems.at[sem(h, d, c)],
                device_id=(dev(d),), device_id_type=pl.DeviceIdType.MESH,
            )

        started = []
        pend = {}

        def start(rdma, d, c):
            rdma.start()
            started.append(rdma)
            pend[(d, c)] = rdma

        for c in range(K):
            for d in range(2):
                r = copy(
                    x_ref.at[pl.ds(my * chunk, chunk), pl.ds(gcol(d, c), sub)],
                    rs_buf(d).at[0, :, pl.ds(c * sub, sub)],
                    0, d, c,
                )
                start(r, d, c)

        for h in range(1, N_DEV - 1):
            for c in range(K):
                for d in range(2):
                    pend[(d, c)].wait_recv()
                    ch = rs_recv_chunk(d, h - 1)
                    rs_buf(d)[h - 1, :, pl.ds(c * sub, sub)] = (
                        rs_buf(d)[h - 1, :, pl.ds(c * sub, sub)]
                        + x_ref[pl.ds(ch * chunk, chunk), pl.ds(gcol(d, c), sub)]
                    )
                    r = copy(
                        rs_buf(d).at[h - 1, :, pl.ds(c * sub, sub)],
                        rs_buf(d).at[h, :, pl.ds(c * sub, sub)],
                        h, d, c,
                    )
                    start(r, d, c)

        for c in range(K):
            for d in range(2):
                pend[(d, c)].wait_recv()
                q = owned_chunk(d)
                out_ref[pl.ds(q * chunk, chunk), pl.ds(gcol(d, c), sub)] = (
                    rs_buf(d)[N_DEV - 2, :, pl.ds(c * sub, sub)]
                    + x_ref[pl.ds(q * chunk, chunk), pl.ds(gcol(d, c), sub)]
                )
                r = copy(
                    out_ref.at[pl.ds(q * chunk, chunk), pl.ds(gcol(d, c), sub)],
                    out_ref.at[pl.ds(q * chunk, chunk), pl.ds(gcol(d, c), sub)],
                    N_DEV - 1, d, c,
                )
                start(r, d, c)

        for t in range(1, N_DEV - 1):
            for c in range(K):
                for d in range(2):
                    pend[(d, c)].wait_recv()
                    ch = ag_send_chunk(d, t)
                    r = copy(
                        out_ref.at[pl.ds(ch * chunk, chunk), pl.ds(gcol(d, c), sub)],
                        out_ref.at[pl.ds(ch * chunk, chunk), pl.ds(gcol(d, c), sub)],
                        N_DEV - 1 + t, d, c,
                    )
                    start(r, d, c)

        for c in range(K):
            for d in range(2):
                pend[(d, c)].wait_recv()
        for r in started:
            r.wait_send()

    return pl.pallas_call(
        body,
        out_shape=jax.ShapeDtypeStruct((m, n), x.dtype),
        in_specs=[pl.BlockSpec(memory_space=pltpu.VMEM)],
        out_specs=pl.BlockSpec(memory_space=pltpu.VMEM),
        scratch_shapes=[
            pltpu.VMEM((N_DEV - 1, chunk, n2), x.dtype),
            pltpu.VMEM((N_DEV - 1, chunk, n2), x.dtype),
            pltpu.SemaphoreType.DMA((n_sems,)),
            pltpu.SemaphoreType.DMA((n_sems,)),
        ],
        compiler_params=pltpu.CompilerParams(collective_id=0),
    )(x)
